# Initial kernel scaffold; baseline (speedup 1.0000x reference)
import jax
import jax.numpy as jnp
from jax.experimental import pallas as pl


def kernel(x, edge_index, W1_0, b1_0, W2_0, b2_0, W1_1, b1_1, W2_1, b2_1, W1_2, b1_2, W2_2, b2_2, Wl1, bl1, Wl2, bl2):
    raise NotImplementedError("write your pallas kernel here")


# speedup vs baseline: 5.5385x; 5.5385x over previous
import functools

import jax
import jax.numpy as jnp
from jax import lax
from jax.experimental import pallas as pl
from jax.experimental.pallas import tpu as pltpu
from jax.experimental.pallas import tpu_sc as plsc

N = 10000
E = 320000
H = 256
HALF = 128
CHUNK = 125
NC = 2
NT = 16
RPT = 624


def _make_agg(split_edges: bool):
  n_chunks = E // CHUNK
  cpt = n_chunks // (NC * NT) if split_edges else n_chunks // NT
  G = 16
  n_groups = cpt // G

  mesh = plsc.VectorSubcoreMesh(core_axis_name="c", subcore_axis_name="s")

  @functools.partial(
      pl.kernel,
      mesh=mesh,
      out_type=jax.ShapeDtypeStruct((NC, N, HALF), jnp.float32),
      scratch_types=[
          pltpu.VMEM_SHARED((N, HALF), jnp.float32),
          pltpu.VMEM((G, CHUNK), jnp.int32),
          pltpu.VMEM((G, CHUNK), jnp.int32),
          pltpu.VMEM((CHUNK, HALF), jnp.float32),
          pltpu.SemaphoreType.DMA,
      ],
  )
  def agg(table, src2, dst2, zeros, out, acc, src_v, dst_v, rows, sem):
    c = lax.axis_index("c")
    s = lax.axis_index("s")
    row0 = s * RPT
    pltpu.sync_copy(zeros.at[pl.ds(0, RPT)], acc.at[pl.ds(row0, RPT)])

    @pl.when(s == NT - 1)
    def _zero_tail():
      pltpu.sync_copy(zeros.at[pl.ds(0, 16)], acc.at[pl.ds(NT * RPT, 16)])

    if split_edges:
      base = (c * NT + s) * cpt
      tab = table
    else:
      base = s * cpt
      tab = table.at[c]
    plsc.subcore_barrier()

    def group(g, carry):
      pltpu.sync_copy(src2.at[pl.ds(base + g * G, G)], src_v)
      pltpu.sync_copy(dst2.at[pl.ds(base + g * G, G)], dst_v)

      def step(j, carry2):
        pltpu.async_copy(tab.at[src_v.at[j]], rows, sem).wait()
        pltpu.sync_copy(rows, acc.at[dst_v.at[j]], add=True)
        return carry2

      lax.fori_loop(0, G, step, 0)
      return carry

    lax.fori_loop(0, n_groups, group, 0)
    plsc.subcore_barrier()
    pltpu.sync_copy(acc.at[pl.ds(row0, RPT)], out.at[c, pl.ds(row0, RPT)])

    @pl.when(s == NT - 1)
    def _copy_tail():
      pltpu.sync_copy(acc.at[pl.ds(NT * RPT, 16)],
                      out.at[c, pl.ds(NT * RPT, 16)])

  return agg


_AGG_EDGE_SPLIT = _make_agg(True)
_AGG_FEAT_SPLIT = _make_agg(False)

BLK = 400


def _mlp0(p, x, W1, b1, W2, b2):
  def body(p_ref, x_ref, w1_ref, b1_ref, w2_ref, b2_ref, o_ref):
    m = p_ref[0] + p_ref[1] + x_ref[...]
    t = jnp.maximum(
        jnp.dot(m, w1_ref[...], preferred_element_type=jnp.float32)
        + b1_ref[...], 0.0)
    o = jnp.maximum(
        jnp.dot(t, w2_ref[...], preferred_element_type=jnp.float32)
        + b2_ref[...], 0.0)
    o_ref[0] = o[:, :HALF]
    o_ref[1] = o[:, HALF:]

  return pl.pallas_call(
      body,
      grid=(N // BLK,),
      in_specs=[
          pl.BlockSpec((2, BLK, HALF), lambda i: (0, i, 0)),
          pl.BlockSpec((BLK, HALF), lambda i: (i, 0)),
          pl.BlockSpec((HALF, H), lambda i: (0, 0)),
          pl.BlockSpec((1, H), lambda i: (0, 0)),
          pl.BlockSpec((H, H), lambda i: (0, 0)),
          pl.BlockSpec((1, H), lambda i: (0, 0)),
      ],
      out_specs=pl.BlockSpec((2, BLK, HALF), lambda i: (0, i, 0)),
      out_shape=jax.ShapeDtypeStruct((2, N, HALF), jnp.float32),
  )(p, x, W1, b1.reshape(1, H), W2, b2.reshape(1, H))


def _mlp_mid(a, h, W1, b1, W2, b2):
  def body(a_ref, h_ref, w1_ref, b1_ref, w2_ref, b2_ref, o_ref):
    m = jnp.concatenate(
        [a_ref[0] + h_ref[0], a_ref[1] + h_ref[1]], axis=1)
    t = jnp.maximum(
        jnp.dot(m, w1_ref[...], preferred_element_type=jnp.float32)
        + b1_ref[...], 0.0)
    o = jnp.maximum(
        jnp.dot(t, w2_ref[...], preferred_element_type=jnp.float32)
        + b2_ref[...], 0.0)
    o_ref[0] = o[:, :HALF]
    o_ref[1] = o[:, HALF:]

  return pl.pallas_call(
      body,
      grid=(N // BLK,),
      in_specs=[
          pl.BlockSpec((2, BLK, HALF), lambda i: (0, i, 0)),
          pl.BlockSpec((2, BLK, HALF), lambda i: (0, i, 0)),
          pl.BlockSpec((H, H), lambda i: (0, 0)),
          pl.BlockSpec((1, H), lambda i: (0, 0)),
          pl.BlockSpec((H, H), lambda i: (0, 0)),
          pl.BlockSpec((1, H), lambda i: (0, 0)),
      ],
      out_specs=pl.BlockSpec((2, BLK, HALF), lambda i: (0, i, 0)),
      out_shape=jax.ShapeDtypeStruct((2, N, HALF), jnp.float32),
  )(a, h, W1, b1.reshape(1, H), W2, b2.reshape(1, H))


def _head(h, Wl1, bl1, Wl2, bl2):
  def body(h_ref, w1_ref, b1_ref, w2_ref, b2_ref, o_ref):
    m = jnp.concatenate([h_ref[0], h_ref[1]], axis=1)
    t = jnp.maximum(
        jnp.dot(m, w1_ref[...], preferred_element_type=jnp.float32)
        + b1_ref[...], 0.0)
    o_ref[...] = (
        jnp.dot(t, w2_ref[...], preferred_element_type=jnp.float32)
        + b2_ref[...])

  return pl.pallas_call(
      body,
      grid=(N // BLK,),
      in_specs=[
          pl.BlockSpec((2, BLK, HALF), lambda i: (0, i, 0)),
          pl.BlockSpec((H, H), lambda i: (0, 0)),
          pl.BlockSpec((1, H), lambda i: (0, 0)),
          pl.BlockSpec((H, 1), lambda i: (0, 0)),
          pl.BlockSpec((1, 1), lambda i: (0, 0)),
      ],
      out_specs=pl.BlockSpec((BLK, 1), lambda i: (i, 0)),
      out_shape=jax.ShapeDtypeStruct((N, 1), jnp.float32),
  )(h, Wl1, bl1.reshape(1, H), Wl2, bl2.reshape(1, 1))


def kernel(x, edge_index, W1_0, b1_0, W2_0, b2_0, W1_1, b1_1, W2_1, b2_1,
           W1_2, b1_2, W2_2, b2_2, Wl1, bl1, Wl2, bl2):
  src2 = edge_index[0].reshape(E // CHUNK, CHUNK)
  dst2 = edge_index[1].reshape(E // CHUNK, CHUNK)
  zeros = jnp.zeros((RPT, HALF), jnp.float32)

  p = _AGG_EDGE_SPLIT(x, src2, dst2, zeros)
  h = _mlp0(p, x, W1_0, b1_0, W2_0, b2_0)
  for (W1, b1, W2, b2) in ((W1_1, b1_1, W2_1, b2_1),
                           (W1_2, b1_2, W2_2, b2_2)):
    a = _AGG_FEAT_SPLIT(h, src2, dst2, zeros)
    h = _mlp_mid(a, h, W1, b1, W2, b2)
  return _head(h, Wl1, bl1, Wl2, bl2)
